# Initial kernel scaffold; baseline (speedup 1.0000x reference)
import jax
import jax.numpy as jnp
from jax.experimental import pallas as pl


def kernel(x, board_table, extra_table):
    raise NotImplementedError("write your pallas kernel here")


# speedup vs baseline: 10.3537x; 10.3537x over previous
import functools

import jax
import jax.numpy as jnp
import numpy as np
from jax import lax
from jax.experimental import pallas as pl
from jax.experimental.pallas import tpu as pltpu
from jax.experimental.pallas import tpu_sc as plsc

B = 16384
H, W = 12, 42
ROW = 2 * H * W
BOARD = H * W
BOARDP = 512
BDIM = 14
EDIM = 7
OL = 5 * 7 * 7
OLP = 248
OE = EDIM * 7
OEP = 56
NW = 32
NB = B // NW
CB = 8
NCHUNK = NB // CB


def _local_perm() -> np.ndarray:
    idx = np.full((256,), BOARD, np.int32)
    for k in range(OL):
        i, r, c = k // 49, (k % 49) // 7, k % 7
        idx[k] = BOARD + r * 42 + 7 * i + c
    return idx


def _extra_consts() -> tuple[np.ndarray, np.ndarray]:
    jv = np.full((64,), BOARD + 35, np.int32)
    dv = np.zeros((64,), np.int32)
    for k in range(OE):
        jv[k] = BOARD + 35 + (k % 7)
        dv[k] = k // 7
    return jv, dv


_LP = _local_perm()
_EJ, _ED = _extra_consts()


@functools.cache
def _build_sc_parse():
    mesh = plsc.VectorSubcoreMesh(core_axis_name="c", subcore_axis_name="s")

    @functools.partial(
        pl.kernel,
        out_type=[
            jax.ShapeDtypeStruct((B, BDIM, BOARD), jnp.float32),
            jax.ShapeDtypeStruct((B, OLP), jnp.float32),
            jax.ShapeDtypeStruct((B, OEP), jnp.float32),
        ],
        mesh=mesh,
        compiler_params=pltpu.CompilerParams(
            needs_layout_passes=False, use_tc_tiling_on_sc=False),
        scratch_types=[
            pltpu.VMEM((CB, ROW), jnp.float32),
            pltpu.VMEM((10, BDIM), jnp.float32),
            pltpu.VMEM((20, EDIM), jnp.float32),
            pltpu.VMEM((256,), jnp.int32),
            pltpu.VMEM((64,), jnp.int32),
            pltpu.VMEM((64,), jnp.int32),
            pltpu.VMEM((CB, BDIM, BOARDP), jnp.float32),
            pltpu.VMEM((CB, 256), jnp.float32),
            pltpu.VMEM((CB, 64), jnp.float32),
        ],
    )
    def _sc_parse(xf, bt, et, lp, ej, ed, og, ol, oe,
                  xin, btv, etv, lpv, ejv, edv, ogb, olb, oeb):
        wid = lax.axis_index("s") * 2 + lax.axis_index("c")
        base0 = wid * NB
        pltpu.sync_copy(bt, btv)
        pltpu.sync_copy(et, etv)
        pltpu.sync_copy(lp, lpv)
        pltpu.sync_copy(ej, ejv)
        pltpu.sync_copy(ed, edv)

        def chunk_body(ci, carry):
            base = base0 + ci * CB
            pltpu.sync_copy(xf.at[pl.ds(base, CB)], xin)

            def batch_body(bi, bcarry):
                bvec = jnp.full((16,), bi, jnp.int32)
                for pv in range(32):
                    bidx = xin[bi, pl.ds(16 * pv, 16)].astype(jnp.int32)
                    for d in range(BDIM):
                        vals = plsc.load_gather(
                            btv, [bidx, jnp.full((16,), d, jnp.int32)])
                        ogb[bi, d, pl.ds(16 * pv, 16)] = vals
                for k in range(16):
                    src = plsc.load_gather(xin, [bvec, lpv[pl.ds(16 * k, 16)]])
                    olb[bi, pl.ds(16 * k, 16)] = src
                for k in range(4):
                    ix = plsc.load_gather(
                        xin, [bvec, ejv[pl.ds(16 * k, 16)]]).astype(jnp.int32)
                    vals = plsc.load_gather(etv, [ix, edv[pl.ds(16 * k, 16)]])
                    oeb[bi, pl.ds(16 * k, 16)] = vals
                return bcarry

            lax.fori_loop(0, CB, batch_body, 0)
            pltpu.sync_copy(ogb.at[:, :, pl.ds(0, BOARD)],
                            og.at[pl.ds(base, CB)])
            pltpu.sync_copy(olb.at[:, pl.ds(0, OLP)], ol.at[pl.ds(base, CB)])
            pltpu.sync_copy(oeb.at[:, pl.ds(0, OEP)], oe.at[pl.ds(base, CB)])
            return carry

        lax.fori_loop(0, NCHUNK, chunk_body, 0)

    return _sc_parse


def kernel(x, board_table, extra_table):
    xf = x.reshape(B, ROW)
    og, ol, oe = _build_sc_parse()(
        xf, board_table, extra_table,
        jnp.asarray(_LP), jnp.asarray(_EJ), jnp.asarray(_ED))
    return (og.reshape(B, BDIM, H, W),
            ol[:, :OL].reshape(B, 5, 7, 7),
            oe[:, :OE].reshape(B, EDIM, 7))


# speedup vs baseline: 17.2680x; 1.6678x over previous
import functools

import jax
import jax.numpy as jnp
import numpy as np
from jax import lax
from jax.experimental import pallas as pl
from jax.experimental.pallas import tpu as pltpu
from jax.experimental.pallas import tpu_sc as plsc

B = 16384
H, W = 12, 42
ROW = 2 * H * W
BOARD = H * W
BOARDP = 512
BDIM = 14
EDIM = 7
OL = 5 * 7 * 7
OLP = 248
OE = EDIM * 7
OEP = 56
NW = 32
NB = B // NW
CB = 8
NCHUNK = NB // CB


def _local_perm() -> np.ndarray:
    idx = np.full((256,), BOARD, np.int32)
    for k in range(OL):
        i, r, c = k // 49, (k % 49) // 7, k % 7
        idx[k] = BOARD + r * 42 + 7 * i + c
    return idx


def _extra_consts() -> tuple[np.ndarray, np.ndarray]:
    jv = np.full((64,), BOARD + 35, np.int32)
    dv = np.zeros((64,), np.int32)
    for k in range(OE):
        jv[k] = BOARD + 35 + (k % 7)
        dv[k] = k // 7
    return jv, dv


_LP = _local_perm()
_EJ, _ED = _extra_consts()


@functools.cache
def _build_sc_parse():
    mesh = plsc.VectorSubcoreMesh(core_axis_name="c", subcore_axis_name="s")

    @functools.partial(
        pl.kernel,
        out_type=[
            jax.ShapeDtypeStruct((B, BDIM, BOARD), jnp.float32),
            jax.ShapeDtypeStruct((B, OLP), jnp.float32),
            jax.ShapeDtypeStruct((B, OEP), jnp.float32),
        ],
        mesh=mesh,
        compiler_params=pltpu.CompilerParams(
            needs_layout_passes=False, use_tc_tiling_on_sc=False),
        scratch_types=[
            pltpu.VMEM((CB, ROW), jnp.float32),
            pltpu.VMEM((10, BDIM), jnp.float32),
            pltpu.VMEM((20, EDIM), jnp.float32),
            pltpu.VMEM((256,), jnp.int32),
            pltpu.VMEM((64,), jnp.int32),
            pltpu.VMEM((64,), jnp.int32),
            pltpu.VMEM((CB, BDIM, BOARDP), jnp.float32),
            pltpu.VMEM((CB, 256), jnp.float32),
            pltpu.VMEM((CB, 64), jnp.float32),
        ],
    )
    def _sc_parse(xf, bt, et, lp, ej, ed, og, ol, oe,
                  xin, btv, etv, lpv, ejv, edv, ogb, olb, oeb):
        wid = lax.axis_index("s") * 2 + lax.axis_index("c")
        base0 = wid * NB
        pltpu.sync_copy(bt, btv)
        pltpu.sync_copy(et, etv)
        pltpu.sync_copy(lp, lpv)
        pltpu.sync_copy(ej, ejv)
        pltpu.sync_copy(ed, edv)

        def chunk_body(ci, carry):
            base = base0 + ci * CB
            pltpu.sync_copy(xf.at[pl.ds(base, CB)], xin)

            @plsc.parallel_loop(0, CB * 32, unroll=2)
            def _board(i):
                bi = i >> 5
                pv = i & 31
                bidx = xin[bi, pl.ds(pv * 16, 16)].astype(jnp.int32)
                for d in range(BDIM):
                    vals = plsc.load_gather(
                        btv, [bidx, jnp.full((16,), d, jnp.int32)])
                    ogb[bi, d, pl.ds(pv * 16, 16)] = vals

            @plsc.parallel_loop(0, CB, unroll=2)
            def _aux(bi):
                bvec = jnp.full((16,), bi, jnp.int32)
                for k in range(16):
                    src = plsc.load_gather(xin, [bvec, lpv[pl.ds(16 * k, 16)]])
                    olb[bi, pl.ds(16 * k, 16)] = src
                for k in range(4):
                    ix = plsc.load_gather(
                        xin, [bvec, ejv[pl.ds(16 * k, 16)]]).astype(jnp.int32)
                    vals = plsc.load_gather(etv, [ix, edv[pl.ds(16 * k, 16)]])
                    oeb[bi, pl.ds(16 * k, 16)] = vals
            pltpu.sync_copy(ogb.at[:, :, pl.ds(0, BOARD)],
                            og.at[pl.ds(base, CB)])
            pltpu.sync_copy(olb.at[:, pl.ds(0, OLP)], ol.at[pl.ds(base, CB)])
            pltpu.sync_copy(oeb.at[:, pl.ds(0, OEP)], oe.at[pl.ds(base, CB)])
            return carry

        lax.fori_loop(0, NCHUNK, chunk_body, 0)

    return _sc_parse


def kernel(x, board_table, extra_table):
    xf = x.reshape(B, ROW)
    og, ol, oe = _build_sc_parse()(
        xf, board_table, extra_table,
        jnp.asarray(_LP), jnp.asarray(_EJ), jnp.asarray(_ED))
    return (og.reshape(B, BDIM, H, W),
            ol[:, :OL].reshape(B, 5, 7, 7),
            oe[:, :OE].reshape(B, EDIM, 7))
